# baseline (device time: 86914 ns/iter reference)
import functools

import jax
import jax.numpy as jnp
from jax import lax
from jax.experimental import pallas as pl
from jax.experimental.pallas import tpu as pltpu

N_DEV = 4
SQ_PER = 256
SQ = 1024
D_MODEL = 1024
H_PER = 8
DH = 128
KV_USED = 1024
BLK = 64
SCALE = 0.08838834764831843


def kernel(x, Wq, K_ext, V_ext, Wo):
    i = lax.axis_index("i")
    xb = x[0].astype(jnp.bfloat16)
    wq = Wq.astype(jnp.bfloat16)
    wo = Wo.astype(jnp.bfloat16)
    k_my = lax.dynamic_slice_in_dim(K_ext[0, :KV_USED], i * H_PER, H_PER, axis=1)
    v_my = lax.dynamic_slice_in_dim(V_ext[0, :KV_USED], i * H_PER, H_PER, axis=1)
    k_my = k_my.transpose(1, 0, 2).astype(jnp.bfloat16)
    v_my = v_my.transpose(1, 0, 2).astype(jnp.bfloat16)

    def body(x_ref, wq_ref, k_ref, v_ref, wo_ref, out_ref,
             comm, q_full, ctx, partial, rs_send, rs_recv,
             ag_send_sems, ag_recv_sems, rs_send_sem, rs_recv_sems):
        my = lax.axis_index("i")
        left = lax.rem(my + N_DEV - 1, N_DEV)
        right = lax.rem(my + 1, N_DEV)

        barrier = pltpu.get_barrier_semaphore()
        for nbr in (left, right):
            pl.semaphore_signal(barrier, inc=1, device_id=(nbr,),
                                device_id_type=pl.DeviceIdType.MESH)
        pl.semaphore_wait(barrier, 2)

        comm[0, :, :] = x_ref[:, :]
        for h in range(N_DEV - 1):
            rdma = pltpu.make_async_remote_copy(
                src_ref=comm.at[h],
                dst_ref=comm.at[h + 1],
                send_sem=ag_send_sems.at[h],
                recv_sem=ag_recv_sems.at[h],
                device_id=(right,),
                device_id_type=pl.DeviceIdType.MESH,
            )
            rdma.start()
            rdma.wait()

        for j in range(N_DEV):
            qblk = jnp.dot(comm[j, :, :], wq_ref[:, :],
                           preferred_element_type=jnp.float32)
            b = lax.rem(my - j + N_DEV, N_DEV)
            q_full[pl.ds(b * SQ_PER, SQ_PER), :] = qblk.astype(jnp.bfloat16)

        qb_idx = lax.broadcasted_iota(jnp.int32, (SQ, KV_USED), 0) // BLK
        kb_idx = lax.broadcasted_iota(jnp.int32, (SQ, KV_USED), 1) // BLK
        mask = kb_idx <= qb_idx
        for h in range(H_PER):
            q_h = q_full[:, h * DH:(h + 1) * DH]
            s = lax.dot_general(q_h, k_ref[h, :, :], (((1,), (1,)), ((), ())),
                                preferred_element_type=jnp.float32) * SCALE
            s = jnp.where(mask, s, -1e9)
            m = jnp.max(s, axis=-1, keepdims=True)
            w = jnp.exp(s - m)
            p = (w / jnp.sum(w, axis=-1, keepdims=True)).astype(jnp.bfloat16)
            c = jnp.dot(p, v_ref[h, :, :], preferred_element_type=jnp.float32)
            ctx[:, h * DH:(h + 1) * DH] = c.astype(jnp.bfloat16)

        partial[:, :] = jnp.dot(ctx[:, :], wo_ref[:, :],
                                preferred_element_type=jnp.float32)

        for st in range(N_DEV - 1):
            b = lax.rem(my - 1 - st + 2 * N_DEV, N_DEV)
            val = partial[pl.ds(b * SQ_PER, SQ_PER), :]
            if st > 0:
                val = val + rs_recv[st - 1, :, :].astype(jnp.float32)
            rs_send[:, :] = val.astype(jnp.bfloat16)
            rdma = pltpu.make_async_remote_copy(
                src_ref=rs_send,
                dst_ref=rs_recv.at[st],
                send_sem=rs_send_sem,
                recv_sem=rs_recv_sems.at[st],
                device_id=(right,),
                device_id_type=pl.DeviceIdType.MESH,
            )
            rdma.start()
            rdma.wait()

        out_ref[:, :] = (rs_recv[N_DEV - 2, :, :].astype(jnp.float32)
                         + partial[pl.ds(my * SQ_PER, SQ_PER), :])

        @functools.partial(pl.run_scoped, sem=pltpu.SemaphoreType.REGULAR)
        def _(sem):
            for nbr in (left, right):
                pl.semaphore_signal(sem, inc=1, device_id=(nbr,),
                                    device_id_type=pl.DeviceIdType.MESH)
            pl.semaphore_wait(sem, 2)

    out = pl.pallas_call(
        body,
        out_shape=jax.ShapeDtypeStruct((SQ_PER, D_MODEL), jnp.float32),
        in_specs=[pl.BlockSpec(memory_space=pltpu.VMEM)] * 5,
        out_specs=pl.BlockSpec(memory_space=pltpu.VMEM),
        scratch_shapes=[
            pltpu.VMEM((N_DEV, SQ_PER, D_MODEL), jnp.bfloat16),
            pltpu.VMEM((SQ, H_PER * DH), jnp.bfloat16),
            pltpu.VMEM((SQ, H_PER * DH), jnp.bfloat16),
            pltpu.VMEM((SQ, D_MODEL), jnp.float32),
            pltpu.VMEM((SQ_PER, D_MODEL), jnp.bfloat16),
            pltpu.VMEM((N_DEV - 1, SQ_PER, D_MODEL), jnp.bfloat16),
            pltpu.SemaphoreType.DMA((N_DEV - 1,)),
            pltpu.SemaphoreType.DMA((N_DEV - 1,)),
            pltpu.SemaphoreType.DMA,
            pltpu.SemaphoreType.DMA((N_DEV - 1,)),
        ],
        compiler_params=pltpu.CompilerParams(collective_id=0),
    )(xb, wq, k_my, v_my, wo)
    return out[None]


# device time: 61521 ns/iter; 1.4128x vs baseline; 1.4128x over previous
import jax
import jax.numpy as jnp
from jax import lax
from jax.experimental import pallas as pl
from jax.experimental.pallas import tpu as pltpu

N_DEV = 4
SQ_PER = 256
SQ = 1024
D_MODEL = 1024
H_PER = 8
DH = 128
KV_USED = 1024
BLK = 64
SCALE = 0.08838834764831843


def kernel(x, Wq, K_ext, V_ext, Wo):
    i = lax.axis_index("i")
    xb = x[0].astype(jnp.bfloat16)
    wq = Wq.astype(jnp.bfloat16)
    wo = Wo.astype(jnp.bfloat16)
    k_my = lax.dynamic_slice_in_dim(K_ext[0, :KV_USED], i * H_PER, H_PER, axis=1)
    v_my = lax.dynamic_slice_in_dim(V_ext[0, :KV_USED], i * H_PER, H_PER, axis=1)
    k_my = k_my.reshape(KV_USED, H_PER * DH).astype(jnp.bfloat16)
    v_my = v_my.reshape(KV_USED, H_PER * DH).astype(jnp.bfloat16)

    def body(x_ref, wq_ref, k_ref, v_ref, wo_ref, out_ref,
             comm, ctx_blk, partial_own, rs_send, rs_recv,
             ag_send_sems, ag_recv_sems, rs_send_sems, rs_recv_sems):
        my = lax.axis_index("i")
        left = lax.rem(my + N_DEV - 1, N_DEV)
        right = lax.rem(my + 1, N_DEV)

        barrier = pltpu.get_barrier_semaphore()
        for nbr in (left, right):
            pl.semaphore_signal(barrier, inc=1, device_id=(nbr,),
                                device_id_type=pl.DeviceIdType.MESH)
        pl.semaphore_wait(barrier, 2)

        row_blk = lax.broadcasted_iota(jnp.int32, (SQ_PER, KV_USED), 0) // BLK
        kb_idx = lax.broadcasted_iota(jnp.int32, (SQ_PER, KV_USED), 1) // BLK

        comm[0, :, :] = x_ref[:, :]
        partial_sends = []
        for j in range(N_DEV):
            ag = None
            if j < N_DEV - 1:
                ag = pltpu.make_async_remote_copy(
                    src_ref=comm.at[j],
                    dst_ref=comm.at[j + 1],
                    send_sem=ag_send_sems.at[j],
                    recv_sem=ag_recv_sems.at[j],
                    device_id=(right,),
                    device_id_type=pl.DeviceIdType.MESH,
                )
                ag.start()

            b = lax.rem(my - j + N_DEV, N_DEV)
            qblk = jnp.dot(comm[j, :, :], wq_ref[:, :],
                           preferred_element_type=jnp.float32
                           ).astype(jnp.bfloat16)
            mask = kb_idx <= b * (SQ_PER // BLK) + row_blk
            for h in range(H_PER):
                q_h = qblk[:, h * DH:(h + 1) * DH]
                k_h = k_ref[:, h * DH:(h + 1) * DH]
                s = lax.dot_general(q_h, k_h, (((1,), (1,)), ((), ())),
                                    preferred_element_type=jnp.float32) * SCALE
                s = jnp.where(mask, s, -1e9)
                m = jnp.max(s, axis=-1, keepdims=True)
                w = jnp.exp(s - m)
                p = (w / jnp.sum(w, axis=-1, keepdims=True)).astype(jnp.bfloat16)
                c = jnp.dot(p, v_ref[:, h * DH:(h + 1) * DH],
                            preferred_element_type=jnp.float32)
                ctx_blk[:, h * DH:(h + 1) * DH] = c.astype(jnp.bfloat16)

            pblk = jnp.dot(ctx_blk[:, :], wo_ref[:, :],
                           preferred_element_type=jnp.float32)
            if j == 0:
                partial_own[:, :] = pblk
            else:
                rs_send[j - 1, :, :] = pblk.astype(jnp.bfloat16)
                ps = pltpu.make_async_remote_copy(
                    src_ref=rs_send.at[j - 1],
                    dst_ref=rs_recv.at[N_DEV - 1 - j],
                    send_sem=rs_send_sems.at[j - 1],
                    recv_sem=rs_recv_sems.at[N_DEV - 1 - j],
                    device_id=(b,),
                    device_id_type=pl.DeviceIdType.MESH,
                )
                ps.start()
                partial_sends.append(ps)

            if ag is not None:
                ag.wait()

        acc = partial_own[:, :]
        for s_ in range(N_DEV - 1):
            recv = pltpu.make_async_remote_copy(
                src_ref=rs_send.at[0],
                dst_ref=rs_recv.at[s_],
                send_sem=rs_send_sems.at[0],
                recv_sem=rs_recv_sems.at[s_],
                device_id=(right,),
                device_id_type=pl.DeviceIdType.MESH,
            )
            recv.wait_recv()
            acc = acc + rs_recv[s_, :, :].astype(jnp.float32)
        out_ref[:, :] = acc
        for ps in partial_sends:
            ps.wait_send()

    out = pl.pallas_call(
        body,
        out_shape=jax.ShapeDtypeStruct((SQ_PER, D_MODEL), jnp.float32),
        in_specs=[pl.BlockSpec(memory_space=pltpu.VMEM)] * 5,
        out_specs=pl.BlockSpec(memory_space=pltpu.VMEM),
        scratch_shapes=[
            pltpu.VMEM((N_DEV, SQ_PER, D_MODEL), jnp.bfloat16),
            pltpu.VMEM((SQ_PER, H_PER * DH), jnp.bfloat16),
            pltpu.VMEM((SQ_PER, D_MODEL), jnp.float32),
            pltpu.VMEM((N_DEV - 1, SQ_PER, D_MODEL), jnp.bfloat16),
            pltpu.VMEM((N_DEV - 1, SQ_PER, D_MODEL), jnp.bfloat16),
            pltpu.SemaphoreType.DMA((N_DEV - 1,)),
            pltpu.SemaphoreType.DMA((N_DEV - 1,)),
            pltpu.SemaphoreType.DMA((N_DEV - 1,)),
            pltpu.SemaphoreType.DMA((N_DEV - 1,)),
        ],
        compiler_params=pltpu.CompilerParams(collective_id=0),
    )(xb, wq, k_my, v_my, wo)
    return out[None]


# device time: 49633 ns/iter; 1.7511x vs baseline; 1.2395x over previous
import jax
import jax.numpy as jnp
from jax import lax
from jax.experimental import pallas as pl
from jax.experimental.pallas import tpu as pltpu

N_DEV = 4
SQ_PER = 256
SQ = 1024
D_MODEL = 1024
H_PER = 8
DH = 128
KV_USED = 1024
BLK = 64
SCALE = 0.08838834764831843


def kernel(x, Wq, K_ext, V_ext, Wo):
    def body(x_ref, wq_ref, k_hbm, v_hbm, wo_ref, out_ref,
             comm, wq_bf, wo_bf, k_f32, v_f32, k_bf, v_bf,
             ctx_blk, partial_own, rs_send, rs_recv,
             kv_sems, ag_send_sems, ag_recv_sems, rs_send_sems, rs_recv_sems):
        my = lax.axis_index("i")
        left = lax.rem(my + N_DEV - 1, N_DEV)
        right = lax.rem(my + 1, N_DEV)

        kv_copies = []
        for h in range(H_PER):
            for t, (hbm, dst) in enumerate(((k_hbm, k_f32), (v_hbm, v_f32))):
                cp = pltpu.make_async_copy(
                    hbm.at[0, pl.ds(0, KV_USED), my * H_PER + h, :],
                    dst.at[:, pl.ds(h * DH, DH)],
                    kv_sems.at[t * H_PER + h],
                )
                cp.start()
                kv_copies.append(cp)

        barrier = pltpu.get_barrier_semaphore()
        for nbr in (left, right):
            pl.semaphore_signal(barrier, inc=1, device_id=(nbr,),
                                device_id_type=pl.DeviceIdType.MESH)
        pl.semaphore_wait(barrier, 2)

        comm[0, :, :] = x_ref[0, :, :].astype(jnp.bfloat16)
        wq_bf[:, :] = wq_ref[:, :].astype(jnp.bfloat16)
        wo_bf[:, :] = wo_ref[:, :].astype(jnp.bfloat16)

        row_blk = lax.broadcasted_iota(jnp.int32, (SQ_PER, KV_USED), 0) // BLK
        kb_idx = lax.broadcasted_iota(jnp.int32, (SQ_PER, KV_USED), 1) // BLK

        partial_sends = []
        for j in range(N_DEV):
            ag = None
            if j < N_DEV - 1:
                ag = pltpu.make_async_remote_copy(
                    src_ref=comm.at[j],
                    dst_ref=comm.at[j + 1],
                    send_sem=ag_send_sems.at[j],
                    recv_sem=ag_recv_sems.at[j],
                    device_id=(right,),
                    device_id_type=pl.DeviceIdType.MESH,
                )
                ag.start()

            b = lax.rem(my - j + N_DEV, N_DEV)
            qblk = jnp.dot(comm[j, :, :], wq_bf[:, :],
                           preferred_element_type=jnp.float32
                           ).astype(jnp.bfloat16)
            if j == 0:
                for cp in kv_copies:
                    cp.wait()
                k_bf[:, :] = k_f32[:, :].astype(jnp.bfloat16)
                v_bf[:, :] = v_f32[:, :].astype(jnp.bfloat16)

            mask = kb_idx <= b * (SQ_PER // BLK) + row_blk
            for h in range(H_PER):
                q_h = qblk[:, h * DH:(h + 1) * DH]
                k_h = k_bf[:, h * DH:(h + 1) * DH]
                s = lax.dot_general(q_h, k_h, (((1,), (1,)), ((), ())),
                                    preferred_element_type=jnp.float32) * SCALE
                s = jnp.where(mask, s, -1e9)
                m = jnp.max(s, axis=-1, keepdims=True)
                w = jnp.exp(s - m)
                p = (w / jnp.sum(w, axis=-1, keepdims=True)).astype(jnp.bfloat16)
                c = jnp.dot(p, v_bf[:, h * DH:(h + 1) * DH],
                            preferred_element_type=jnp.float32)
                ctx_blk[:, h * DH:(h + 1) * DH] = c.astype(jnp.bfloat16)

            pblk = jnp.dot(ctx_blk[:, :], wo_bf[:, :],
                           preferred_element_type=jnp.float32)
            if j == 0:
                partial_own[:, :] = pblk
            else:
                rs_send[j - 1, :, :] = pblk.astype(jnp.bfloat16)
                ps = pltpu.make_async_remote_copy(
                    src_ref=rs_send.at[j - 1],
                    dst_ref=rs_recv.at[N_DEV - 1 - j],
                    send_sem=rs_send_sems.at[j - 1],
                    recv_sem=rs_recv_sems.at[N_DEV - 1 - j],
                    device_id=(b,),
                    device_id_type=pl.DeviceIdType.MESH,
                )
                ps.start()
                partial_sends.append(ps)

            if ag is not None:
                ag.wait()

        acc = partial_own[:, :]
        for s_ in range(N_DEV - 1):
            recv = pltpu.make_async_remote_copy(
                src_ref=rs_send.at[0],
                dst_ref=rs_recv.at[s_],
                send_sem=rs_send_sems.at[0],
                recv_sem=rs_recv_sems.at[s_],
                device_id=(right,),
                device_id_type=pl.DeviceIdType.MESH,
            )
            recv.wait_recv()
            acc = acc + rs_recv[s_, :, :].astype(jnp.float32)
        out_ref[0, :, :] = acc
        for ps in partial_sends:
            ps.wait_send()

    return pl.pallas_call(
        body,
        out_shape=jax.ShapeDtypeStruct((1, SQ_PER, D_MODEL), jnp.float32),
        in_specs=[
            pl.BlockSpec(memory_space=pltpu.VMEM),
            pl.BlockSpec(memory_space=pltpu.VMEM),
            pl.BlockSpec(memory_space=pl.ANY),
            pl.BlockSpec(memory_space=pl.ANY),
            pl.BlockSpec(memory_space=pltpu.VMEM),
        ],
        out_specs=pl.BlockSpec(memory_space=pltpu.VMEM),
        scratch_shapes=[
            pltpu.VMEM((N_DEV, SQ_PER, D_MODEL), jnp.bfloat16),
            pltpu.VMEM((D_MODEL, H_PER * DH), jnp.bfloat16),
            pltpu.VMEM((H_PER * DH, D_MODEL), jnp.bfloat16),
            pltpu.VMEM((KV_USED, H_PER * DH), jnp.float32),
            pltpu.VMEM((KV_USED, H_PER * DH), jnp.float32),
            pltpu.VMEM((KV_USED, H_PER * DH), jnp.bfloat16),
            pltpu.VMEM((KV_USED, H_PER * DH), jnp.bfloat16),
            pltpu.VMEM((SQ_PER, H_PER * DH), jnp.bfloat16),
            pltpu.VMEM((SQ_PER, D_MODEL), jnp.float32),
            pltpu.VMEM((N_DEV - 1, SQ_PER, D_MODEL), jnp.bfloat16),
            pltpu.VMEM((N_DEV - 1, SQ_PER, D_MODEL), jnp.bfloat16),
            pltpu.SemaphoreType.DMA((2 * H_PER,)),
            pltpu.SemaphoreType.DMA((N_DEV - 1,)),
            pltpu.SemaphoreType.DMA((N_DEV - 1,)),
            pltpu.SemaphoreType.DMA((N_DEV - 1,)),
            pltpu.SemaphoreType.DMA((N_DEV - 1,)),
        ],
        compiler_params=pltpu.CompilerParams(collective_id=0),
    )(x, Wq, K_ext, V_ext, Wo)


# device time: 45690 ns/iter; 1.9023x vs baseline; 1.0863x over previous
import functools

import jax
import jax.numpy as jnp
from jax import lax
from jax.experimental import pallas as pl
from jax.experimental.pallas import tpu as pltpu

N_DEV = 4
SQ_PER = 256
SQ = 1024
D_MODEL = 1024
H_PER = 8
DH = 128
KV_USED = 1024
BLK = 64
SCALE = 0.08838834764831843


def kernel(x, Wq, K_ext, V_ext, Wo):
    def body(x_ref, wq_ref, k_hbm, v_hbm, wo_ref, out_ref,
             comm, wq_bf, wo_bf, k_f32, v_f32, k_bf, v_bf,
             ctx_blk, acc_all, l_all, partial_own, rs_send, rs_recv,
             kv_sems, ag_send_sems, ag_recv_sems, rs_send_sems, rs_recv_sems):
        my = lax.axis_index("i")
        left = lax.rem(my + N_DEV - 1, N_DEV)
        right = lax.rem(my + 1, N_DEV)

        kv_copies = []
        for h in range(H_PER):
            for t, (hbm, dst) in enumerate(((k_hbm, k_f32), (v_hbm, v_f32))):
                cp = pltpu.make_async_copy(
                    hbm.at[0, pl.ds(0, KV_USED), my * H_PER + h, :],
                    dst.at[:, pl.ds(h * DH, DH)],
                    kv_sems.at[t * H_PER + h],
                )
                cp.start()
                kv_copies.append(cp)

        barrier = pltpu.get_barrier_semaphore()
        for nbr in (left, right):
            pl.semaphore_signal(barrier, inc=1, device_id=(nbr,),
                                device_id_type=pl.DeviceIdType.MESH)
        pl.semaphore_wait(barrier, 2)

        comm[0, :, :] = x_ref[0, :, :].astype(jnp.bfloat16)
        wq_bf[:, :] = wq_ref[:, :].astype(jnp.bfloat16)
        wo_bf[:, :] = wo_ref[:, :].astype(jnp.bfloat16)

        row_blk = lax.broadcasted_iota(jnp.int32, (SQ_PER, SQ_PER), 0) // BLK
        kb_loc = lax.broadcasted_iota(jnp.int32, (SQ_PER, SQ_PER), 1) // BLK

        partial_sends = []
        for j in range(N_DEV):
            ag = None
            if j < N_DEV - 1:
                ag = pltpu.make_async_remote_copy(
                    src_ref=comm.at[j],
                    dst_ref=comm.at[j + 1],
                    send_sem=ag_send_sems.at[j],
                    recv_sem=ag_recv_sems.at[j],
                    device_id=(right,),
                    device_id_type=pl.DeviceIdType.MESH,
                )
                ag.start()

            b = lax.rem(my - j + N_DEV, N_DEV)
            qblk = jnp.dot(comm[j, :, :], wq_bf[:, :],
                           preferred_element_type=jnp.float32
                           ).astype(jnp.bfloat16)
            if j == 0:
                for cp in kv_copies:
                    cp.wait()
                k_bf[:, :] = k_f32[:, :].astype(jnp.bfloat16)
                v_bf[:, :] = v_f32[:, :].astype(jnp.bfloat16)

            acc_all[:, :] = jnp.zeros((SQ_PER, H_PER * DH), jnp.float32)
            l_all[:, :] = jnp.zeros((SQ_PER, H_PER), jnp.float32)

            def _chunk(c):
                mask = kb_loc + c * (SQ_PER // BLK) <= b * (SQ_PER // BLK) + row_blk
                ks = slice(c * SQ_PER, (c + 1) * SQ_PER)
                for h in range(H_PER):
                    hs = slice(h * DH, (h + 1) * DH)
                    q_h = qblk[:, hs]
                    s = lax.dot_general(q_h, k_bf[ks, hs],
                                        (((1,), (1,)), ((), ())),
                                        preferred_element_type=jnp.float32)
                    w = jnp.where(mask, jnp.exp(s * SCALE), 0.0)
                    l_all[:, h:h + 1] = (l_all[:, h:h + 1]
                                         + jnp.sum(w, axis=-1, keepdims=True))
                    acc_all[:, hs] = acc_all[:, hs] + jnp.dot(
                        w.astype(jnp.bfloat16), v_bf[ks, hs],
                        preferred_element_type=jnp.float32)

            _chunk(0)
            for c in range(1, N_DEV):
                pl.when(c <= b)(functools.partial(_chunk, c))

            for h in range(H_PER):
                hs = slice(h * DH, (h + 1) * DH)
                inv = 1.0 / l_all[:, h:h + 1]
                ctx_blk[:, hs] = (acc_all[:, hs] * inv).astype(jnp.bfloat16)

            pblk = jnp.dot(ctx_blk[:, :], wo_bf[:, :],
                           preferred_element_type=jnp.float32)
            if j == 0:
                partial_own[:, :] = pblk
            else:
                rs_send[j - 1, :, :] = pblk.astype(jnp.bfloat16)
                ps = pltpu.make_async_remote_copy(
                    src_ref=rs_send.at[j - 1],
                    dst_ref=rs_recv.at[N_DEV - 1 - j],
                    send_sem=rs_send_sems.at[j - 1],
                    recv_sem=rs_recv_sems.at[N_DEV - 1 - j],
                    device_id=(b,),
                    device_id_type=pl.DeviceIdType.MESH,
                )
                ps.start()
                partial_sends.append(ps)

            if ag is not None:
                ag.wait()

        acc = partial_own[:, :]
        for s_ in range(N_DEV - 1):
            recv = pltpu.make_async_remote_copy(
                src_ref=rs_send.at[0],
                dst_ref=rs_recv.at[s_],
                send_sem=rs_send_sems.at[0],
                recv_sem=rs_recv_sems.at[s_],
                device_id=(right,),
                device_id_type=pl.DeviceIdType.MESH,
            )
            recv.wait_recv()
            acc = acc + rs_recv[s_, :, :].astype(jnp.float32)
        out_ref[0, :, :] = acc
        for ps in partial_sends:
            ps.wait_send()

    return pl.pallas_call(
        body,
        out_shape=jax.ShapeDtypeStruct((1, SQ_PER, D_MODEL), jnp.float32),
        in_specs=[
            pl.BlockSpec(memory_space=pltpu.VMEM),
            pl.BlockSpec(memory_space=pltpu.VMEM),
            pl.BlockSpec(memory_space=pl.ANY),
            pl.BlockSpec(memory_space=pl.ANY),
            pl.BlockSpec(memory_space=pltpu.VMEM),
        ],
        out_specs=pl.BlockSpec(memory_space=pltpu.VMEM),
        scratch_shapes=[
            pltpu.VMEM((N_DEV, SQ_PER, D_MODEL), jnp.bfloat16),
            pltpu.VMEM((D_MODEL, H_PER * DH), jnp.bfloat16),
            pltpu.VMEM((H_PER * DH, D_MODEL), jnp.bfloat16),
            pltpu.VMEM((KV_USED, H_PER * DH), jnp.float32),
            pltpu.VMEM((KV_USED, H_PER * DH), jnp.float32),
            pltpu.VMEM((KV_USED, H_PER * DH), jnp.bfloat16),
            pltpu.VMEM((KV_USED, H_PER * DH), jnp.bfloat16),
            pltpu.VMEM((SQ_PER, H_PER * DH), jnp.bfloat16),
            pltpu.VMEM((SQ_PER, H_PER * DH), jnp.float32),
            pltpu.VMEM((SQ_PER, H_PER), jnp.float32),
            pltpu.VMEM((SQ_PER, D_MODEL), jnp.float32),
            pltpu.VMEM((N_DEV - 1, SQ_PER, D_MODEL), jnp.bfloat16),
            pltpu.VMEM((N_DEV - 1, SQ_PER, D_MODEL), jnp.bfloat16),
            pltpu.SemaphoreType.DMA((2 * H_PER,)),
            pltpu.SemaphoreType.DMA((N_DEV - 1,)),
            pltpu.SemaphoreType.DMA((N_DEV - 1,)),
            pltpu.SemaphoreType.DMA((N_DEV - 1,)),
            pltpu.SemaphoreType.DMA((N_DEV - 1,)),
        ],
        compiler_params=pltpu.CompilerParams(collective_id=0),
    )(x, Wq, K_ext, V_ext, Wo)


# device time: 44187 ns/iter; 1.9670x vs baseline; 1.0340x over previous
import functools

import jax
import jax.numpy as jnp
from jax import lax
from jax.experimental import pallas as pl
from jax.experimental.pallas import tpu as pltpu

N_DEV = 4
SQ_PER = 256
SQ = 1024
D_MODEL = 1024
H_PER = 8
DH = 128
KV_USED = 1024
BLK = 64
SCALE = 0.08838834764831843


def kernel(x, Wq, K_ext, V_ext, Wo):
    def body(x_ref, wq_ref, k_hbm, v_hbm, wo_ref, out_ref,
             comm, wq_bf, wo_bf, k_f32, v_f32, k_bf, v_bf,
             ctx_blk, acc_all, l_all, q_own, rs_send, rs_recv,
             kv_sems, ag_send_sems, ag_recv_sems, rs_send_sems, rs_recv_sems):
        my = lax.axis_index("i")
        left = lax.rem(my + N_DEV - 1, N_DEV)
        right = lax.rem(my + 1, N_DEV)

        kv_copies = []
        for h in range(H_PER):
            for t, (hbm, dst) in enumerate(((k_hbm, k_f32), (v_hbm, v_f32))):
                cp = pltpu.make_async_copy(
                    hbm.at[0, pl.ds(0, KV_USED), my * H_PER + h, :],
                    dst.at[:, pl.ds(h * DH, DH)],
                    kv_sems.at[t * H_PER + h],
                )
                cp.start()
                kv_copies.append(cp)

        barrier = pltpu.get_barrier_semaphore()
        for nbr in (left, right):
            pl.semaphore_signal(barrier, inc=1, device_id=(nbr,),
                                device_id_type=pl.DeviceIdType.MESH)
        pl.semaphore_wait(barrier, 2)

        comm[0, :, :] = x_ref[0, :, :].astype(jnp.bfloat16)

        ag0 = pltpu.make_async_remote_copy(
            src_ref=comm.at[0],
            dst_ref=comm.at[1],
            send_sem=ag_send_sems.at[0],
            recv_sem=ag_recv_sems.at[0],
            device_id=(right,),
            device_id_type=pl.DeviceIdType.MESH,
        )
        ag0.start()

        wq_bf[:, :] = wq_ref[:, :].astype(jnp.bfloat16)
        wo_bf[:, :] = wo_ref[:, :].astype(jnp.bfloat16)
        q_own[:, :] = jnp.dot(comm[0, :, :], wq_bf[:, :],
                              preferred_element_type=jnp.float32
                              ).astype(jnp.bfloat16)
        for cp in kv_copies:
            cp.wait()
        k_bf[:, :] = k_f32[:, :].astype(jnp.bfloat16)
        v_bf[:, :] = v_f32[:, :].astype(jnp.bfloat16)
        ag0.wait()

        row_blk = lax.broadcasted_iota(jnp.int32, (SQ_PER, SQ_PER), 0) // BLK
        kb_loc = lax.broadcasted_iota(jnp.int32, (SQ_PER, SQ_PER), 1) // BLK

        def attend(qblk, b):
            acc_all[:, :] = jnp.zeros((SQ_PER, H_PER * DH), jnp.float32)
            l_all[:, :] = jnp.zeros((SQ_PER, H_PER), jnp.float32)

            def _chunk(c):
                mask = kb_loc + c * (SQ_PER // BLK) <= b * (SQ_PER // BLK) + row_blk
                ks = slice(c * SQ_PER, (c + 1) * SQ_PER)
                for h in range(H_PER):
                    hs = slice(h * DH, (h + 1) * DH)
                    q_h = qblk[:, hs]
                    s = lax.dot_general(q_h, k_bf[ks, hs],
                                        (((1,), (1,)), ((), ())),
                                        preferred_element_type=jnp.float32)
                    w = jnp.where(mask, jnp.exp(s * SCALE), 0.0)
                    l_all[:, h:h + 1] = (l_all[:, h:h + 1]
                                         + jnp.sum(w, axis=-1, keepdims=True))
                    acc_all[:, hs] = acc_all[:, hs] + jnp.dot(
                        w.astype(jnp.bfloat16), v_bf[ks, hs],
                        preferred_element_type=jnp.float32)

            _chunk(0)
            for c in range(1, N_DEV):
                pl.when(c <= b)(functools.partial(_chunk, c))

            for h in range(H_PER):
                hs = slice(h * DH, (h + 1) * DH)
                inv = 1.0 / l_all[:, h:h + 1]
                ctx_blk[:, hs] = (acc_all[:, hs] * inv).astype(jnp.bfloat16)
            return jnp.dot(ctx_blk[:, :], wo_bf[:, :],
                           preferred_element_type=jnp.float32)

        partial_sends = []
        for j in range(1, N_DEV):
            ag = None
            if j < N_DEV - 1:
                ag = pltpu.make_async_remote_copy(
                    src_ref=comm.at[j],
                    dst_ref=comm.at[j + 1],
                    send_sem=ag_send_sems.at[j],
                    recv_sem=ag_recv_sems.at[j],
                    device_id=(right,),
                    device_id_type=pl.DeviceIdType.MESH,
                )
                ag.start()

            b = lax.rem(my - j + N_DEV, N_DEV)
            qblk = jnp.dot(comm[j, :, :], wq_bf[:, :],
                           preferred_element_type=jnp.float32
                           ).astype(jnp.bfloat16)
            pblk = attend(qblk, b)
            rs_send[j - 1, :, :] = pblk.astype(jnp.bfloat16)
            ps = pltpu.make_async_remote_copy(
                src_ref=rs_send.at[j - 1],
                dst_ref=rs_recv.at[N_DEV - 1 - j],
                send_sem=rs_send_sems.at[j - 1],
                recv_sem=rs_recv_sems.at[N_DEV - 1 - j],
                device_id=(b,),
                device_id_type=pl.DeviceIdType.MESH,
            )
            ps.start()
            partial_sends.append(ps)

            if ag is not None:
                ag.wait()

        acc = attend(q_own[:, :], my)

        for s_ in range(N_DEV - 1):
            recv = pltpu.make_async_remote_copy(
                src_ref=rs_send.at[0],
                dst_ref=rs_recv.at[s_],
                send_sem=rs_send_sems.at[0],
                recv_sem=rs_recv_sems.at[s_],
                device_id=(right,),
                device_id_type=pl.DeviceIdType.MESH,
            )
            recv.wait_recv()
            acc = acc + rs_recv[s_, :, :].astype(jnp.float32)
        out_ref[0, :, :] = acc
        for ps in partial_sends:
            ps.wait_send()

    return pl.pallas_call(
        body,
        out_shape=jax.ShapeDtypeStruct((1, SQ_PER, D_MODEL), jnp.float32),
        in_specs=[
            pl.BlockSpec(memory_space=pltpu.VMEM),
            pl.BlockSpec(memory_space=pltpu.VMEM),
            pl.BlockSpec(memory_space=pl.ANY),
            pl.BlockSpec(memory_space=pl.ANY),
            pl.BlockSpec(memory_space=pltpu.VMEM),
        ],
        out_specs=pl.BlockSpec(memory_space=pltpu.VMEM),
        scratch_shapes=[
            pltpu.VMEM((N_DEV, SQ_PER, D_MODEL), jnp.bfloat16),
            pltpu.VMEM((D_MODEL, H_PER * DH), jnp.bfloat16),
            pltpu.VMEM((H_PER * DH, D_MODEL), jnp.bfloat16),
            pltpu.VMEM((KV_USED, H_PER * DH), jnp.float32),
            pltpu.VMEM((KV_USED, H_PER * DH), jnp.float32),
            pltpu.VMEM((KV_USED, H_PER * DH), jnp.bfloat16),
            pltpu.VMEM((KV_USED, H_PER * DH), jnp.bfloat16),
            pltpu.VMEM((SQ_PER, H_PER * DH), jnp.bfloat16),
            pltpu.VMEM((SQ_PER, H_PER * DH), jnp.float32),
            pltpu.VMEM((SQ_PER, H_PER), jnp.float32),
            pltpu.VMEM((SQ_PER, H_PER * DH), jnp.bfloat16),
            pltpu.VMEM((N_DEV - 1, SQ_PER, D_MODEL), jnp.bfloat16),
            pltpu.VMEM((N_DEV - 1, SQ_PER, D_MODEL), jnp.bfloat16),
            pltpu.SemaphoreType.DMA((2 * H_PER,)),
            pltpu.SemaphoreType.DMA((N_DEV - 1,)),
            pltpu.SemaphoreType.DMA((N_DEV - 1,)),
            pltpu.SemaphoreType.DMA((N_DEV - 1,)),
            pltpu.SemaphoreType.DMA((N_DEV - 1,)),
        ],
        compiler_params=pltpu.CompilerParams(collective_id=0),
    )(x, Wq, K_ext, V_ext, Wo)


# device time: 41429 ns/iter; 2.0979x vs baseline; 1.0666x over previous
import functools

import jax
import jax.numpy as jnp
from jax import lax
from jax.experimental import pallas as pl
from jax.experimental.pallas import tpu as pltpu

N_DEV = 4
SQ_PER = 256
SQ = 1024
D_MODEL = 1024
H_PER = 8
DH = 128
KV_USED = 1024
BLK = 64
SCALE = 0.08838834764831843


def kernel(x, Wq, K_ext, V_ext, Wo):
    def body(x_ref, wq_hbm, k_hbm, v_hbm, wo_hbm, out_ref,
             comm, wq_f32, wo_f32, wq_bf, wo_bf, k_f32, v_f32, k_bf, v_bf,
             ctx_blk, acc_all, l_all, q_own, rs_send, rs_recv,
             kv_sems, w_sems, ag_send_sems, ag_recv_sems,
             rs_send_sems, rs_recv_sems):
        my = lax.axis_index("i")
        left = lax.rem(my + N_DEV - 1, N_DEV)
        right = lax.rem(my + 1, N_DEV)

        wq_cp = pltpu.make_async_copy(wq_hbm, wq_f32, w_sems.at[0])
        wo_cp = pltpu.make_async_copy(wo_hbm, wo_f32, w_sems.at[1])
        wq_cp.start()
        wo_cp.start()
        kv_copies = []
        for h in range(H_PER):
            for t, (hbm, dst) in enumerate(((k_hbm, k_f32), (v_hbm, v_f32))):
                cp = pltpu.make_async_copy(
                    hbm.at[0, pl.ds(0, KV_USED), my * H_PER + h, :],
                    dst.at[:, pl.ds(h * DH, DH)],
                    kv_sems.at[t * H_PER + h],
                )
                cp.start()
                kv_copies.append(cp)

        barrier = pltpu.get_barrier_semaphore()
        for nbr in (left, right):
            pl.semaphore_signal(barrier, inc=1, device_id=(nbr,),
                                device_id_type=pl.DeviceIdType.MESH)
        pl.semaphore_wait(barrier, 2)

        comm[0, :, :] = x_ref[0, :, :].astype(jnp.bfloat16)

        ag0 = pltpu.make_async_remote_copy(
            src_ref=comm.at[0],
            dst_ref=comm.at[1],
            send_sem=ag_send_sems.at[0],
            recv_sem=ag_recv_sems.at[0],
            device_id=(right,),
            device_id_type=pl.DeviceIdType.MESH,
        )
        ag0.start()

        wq_cp.wait()
        wq_bf[:, :] = wq_f32[:, :].astype(jnp.bfloat16)
        q_own[:, :] = jnp.dot(comm[0, :, :], wq_bf[:, :],
                              preferred_element_type=jnp.float32
                              ).astype(jnp.bfloat16)
        wo_cp.wait()
        wo_bf[:, :] = wo_f32[:, :].astype(jnp.bfloat16)
        for cp in kv_copies:
            cp.wait()
        k_bf[:, :] = k_f32[:, :].astype(jnp.bfloat16)
        v_bf[:, :] = v_f32[:, :].astype(jnp.bfloat16)
        ag0.wait()

        row_blk = lax.broadcasted_iota(jnp.int32, (SQ_PER, SQ_PER), 0) // BLK
        kb_loc = lax.broadcasted_iota(jnp.int32, (SQ_PER, SQ_PER), 1) // BLK

        def attend(qblk, b):
            acc_all[:, :] = jnp.zeros((SQ_PER, H_PER * DH), jnp.float32)
            l_all[:, :] = jnp.zeros((SQ_PER, H_PER), jnp.float32)

            def _chunk(c):
                mask = kb_loc + c * (SQ_PER // BLK) <= b * (SQ_PER // BLK) + row_blk
                ks = slice(c * SQ_PER, (c + 1) * SQ_PER)
                for h in range(H_PER):
                    hs = slice(h * DH, (h + 1) * DH)
                    q_h = qblk[:, hs]
                    s = lax.dot_general(q_h, k_bf[ks, hs],
                                        (((1,), (1,)), ((), ())),
                                        preferred_element_type=jnp.float32)
                    w = jnp.where(mask, jnp.exp(s * SCALE), 0.0)
                    l_all[:, h:h + 1] = (l_all[:, h:h + 1]
                                         + jnp.sum(w, axis=-1, keepdims=True))
                    acc_all[:, hs] = acc_all[:, hs] + jnp.dot(
                        w.astype(jnp.bfloat16), v_bf[ks, hs],
                        preferred_element_type=jnp.float32)

            _chunk(0)
            for c in range(1, N_DEV):
                pl.when(c <= b)(functools.partial(_chunk, c))

            for h in range(H_PER):
                hs = slice(h * DH, (h + 1) * DH)
                inv = 1.0 / l_all[:, h:h + 1]
                ctx_blk[:, hs] = (acc_all[:, hs] * inv).astype(jnp.bfloat16)
            return jnp.dot(ctx_blk[:, :], wo_bf[:, :],
                           preferred_element_type=jnp.float32)

        partial_sends = []
        for j in range(1, N_DEV):
            ag = None
            if j < N_DEV - 1:
                ag = pltpu.make_async_remote_copy(
                    src_ref=comm.at[j],
                    dst_ref=comm.at[j + 1],
                    send_sem=ag_send_sems.at[j],
                    recv_sem=ag_recv_sems.at[j],
                    device_id=(right,),
                    device_id_type=pl.DeviceIdType.MESH,
                )
                ag.start()

            b = lax.rem(my - j + N_DEV, N_DEV)
            qblk = jnp.dot(comm[j, :, :], wq_bf[:, :],
                           preferred_element_type=jnp.float32
                           ).astype(jnp.bfloat16)
            pblk = attend(qblk, b)
            rs_send[j - 1, :, :] = pblk.astype(jnp.bfloat16)
            ps = pltpu.make_async_remote_copy(
                src_ref=rs_send.at[j - 1],
                dst_ref=rs_recv.at[N_DEV - 1 - j],
                send_sem=rs_send_sems.at[j - 1],
                recv_sem=rs_recv_sems.at[N_DEV - 1 - j],
                device_id=(b,),
                device_id_type=pl.DeviceIdType.MESH,
            )
            ps.start()
            partial_sends.append(ps)

            if ag is not None:
                ag.wait()

        acc = attend(q_own[:, :], my)

        for s_ in range(N_DEV - 1):
            recv = pltpu.make_async_remote_copy(
                src_ref=rs_send.at[0],
                dst_ref=rs_recv.at[s_],
                send_sem=rs_send_sems.at[0],
                recv_sem=rs_recv_sems.at[s_],
                device_id=(right,),
                device_id_type=pl.DeviceIdType.MESH,
            )
            recv.wait_recv()
            acc = acc + rs_recv[s_, :, :].astype(jnp.float32)
        out_ref[0, :, :] = acc
        for ps in partial_sends:
            ps.wait_send()

    return pl.pallas_call(
        body,
        out_shape=jax.ShapeDtypeStruct((1, SQ_PER, D_MODEL), jnp.float32),
        in_specs=[
            pl.BlockSpec(memory_space=pltpu.VMEM),
            pl.BlockSpec(memory_space=pl.ANY),
            pl.BlockSpec(memory_space=pl.ANY),
            pl.BlockSpec(memory_space=pl.ANY),
            pl.BlockSpec(memory_space=pl.ANY),
        ],
        out_specs=pl.BlockSpec(memory_space=pltpu.VMEM),
        scratch_shapes=[
            pltpu.VMEM((N_DEV, SQ_PER, D_MODEL), jnp.bfloat16),
            pltpu.VMEM((D_MODEL, H_PER * DH), jnp.float32),
            pltpu.VMEM((H_PER * DH, D_MODEL), jnp.float32),
            pltpu.VMEM((D_MODEL, H_PER * DH), jnp.bfloat16),
            pltpu.VMEM((H_PER * DH, D_MODEL), jnp.bfloat16),
            pltpu.VMEM((KV_USED, H_PER * DH), jnp.float32),
            pltpu.VMEM((KV_USED, H_PER * DH), jnp.float32),
            pltpu.VMEM((KV_USED, H_PER * DH), jnp.bfloat16),
            pltpu.VMEM((KV_USED, H_PER * DH), jnp.bfloat16),
            pltpu.VMEM((SQ_PER, H_PER * DH), jnp.bfloat16),
            pltpu.VMEM((SQ_PER, H_PER * DH), jnp.float32),
            pltpu.VMEM((SQ_PER, H_PER), jnp.float32),
            pltpu.VMEM((SQ_PER, H_PER * DH), jnp.bfloat16),
            pltpu.VMEM((N_DEV - 1, SQ_PER, D_MODEL), jnp.bfloat16),
            pltpu.VMEM((N_DEV - 1, SQ_PER, D_MODEL), jnp.bfloat16),
            pltpu.SemaphoreType.DMA((2 * H_PER,)),
            pltpu.SemaphoreType.DMA((2,)),
            pltpu.SemaphoreType.DMA((N_DEV - 1,)),
            pltpu.SemaphoreType.DMA((N_DEV - 1,)),
            pltpu.SemaphoreType.DMA((N_DEV - 1,)),
            pltpu.SemaphoreType.DMA((N_DEV - 1,)),
        ],
        compiler_params=pltpu.CompilerParams(
            collective_id=0, vmem_limit_bytes=64 * 1024 * 1024),
    )(x, Wq, K_ext, V_ext, Wo)
